# baseline (speedup 1.0000x reference)
import jax
import jax.numpy as jnp
from jax.experimental import pallas as pl
from jax.experimental.pallas import tpu as pltpu

NBITS = 256
HIDDEN = 256
SPATIAL = 32
B = 128
C = 128
HW = SPATIAL * SPATIAL

NB = 4
RING = 2
GRID = B // NB


def _aux_body(msg_ref, we_ref, wo_ref, aux_ref):
    we = we_ref[...]
    d = wo_ref[...] - we
    base = jnp.sum(we, axis=0, keepdims=True)
    aux_ref[...] = jax.lax.dot_general(
        msg_ref[...], d, (((1,), (0,)), ((), ())),
        preferred_element_type=jnp.float32) + base


def _out_copies(lat_hbm, out_hbm, tile_ref, lat_sem, tile_sem, g, slot):
    lat_cp = pltpu.make_async_copy(
        lat_hbm.at[pl.ds(g * NB, NB)],
        out_hbm.at[pl.ds(g * NB, NB), pl.ds(0, C)],
        lat_sem.at[slot])
    tile_cp = pltpu.make_async_copy(
        tile_ref.at[slot],
        out_hbm.at[pl.ds(g * NB, NB), pl.ds(C, HIDDEN)],
        tile_sem.at[slot])
    return lat_cp, tile_cp


def _bcast_body(aux_ref, lat_hbm, out_hbm, tile_ref, lat_sem, tile_sem):
    g = pl.program_id(0)
    slot = jax.lax.rem(g, RING)

    @pl.when(g >= RING)
    def _wait_prev():
        lat_cp, tile_cp = _out_copies(
            lat_hbm, out_hbm, tile_ref, lat_sem, tile_sem, g - RING, slot)
        lat_cp.wait()
        tile_cp.wait()

    tile_ref[slot] = jnp.broadcast_to(aux_ref[...], (NB, HIDDEN, HW))

    lat_cp, tile_cp = _out_copies(
        lat_hbm, out_hbm, tile_ref, lat_sem, tile_sem, g, slot)
    lat_cp.start()
    tile_cp.start()

    @pl.when(g == GRID - 1)
    def _drain():
        for r in range(RING):
            gr = GRID - RING + r
            lat_cp, tile_cp = _out_copies(
                lat_hbm, out_hbm, tile_ref, lat_sem, tile_sem,
                gr, jax.lax.rem(jnp.int32(gr), RING))
            lat_cp.wait()
            tile_cp.wait()


def kernel(latents, msg, W):
    lat3 = latents.reshape(B, C, HW)
    msg_f = msg.astype(jnp.float32)
    we = W[0::2]
    wo = W[1::2]
    aux = pl.pallas_call(
        _aux_body,
        in_specs=[
            pl.BlockSpec((B, NBITS), lambda: (0, 0)),
            pl.BlockSpec((NBITS, HIDDEN), lambda: (0, 0)),
            pl.BlockSpec((NBITS, HIDDEN), lambda: (0, 0)),
        ],
        out_specs=pl.BlockSpec((B, HIDDEN), lambda: (0, 0)),
        out_shape=jax.ShapeDtypeStruct((B, HIDDEN), jnp.float32),
    )(msg_f, we, wo)
    aux3 = aux.reshape(B, HIDDEN, 1)
    out = pl.pallas_call(
        _bcast_body,
        grid=(GRID,),
        in_specs=[
            pl.BlockSpec((NB, HIDDEN, 1), lambda g: (g, 0, 0)),
            pl.BlockSpec(memory_space=pl.ANY),
        ],
        out_specs=pl.BlockSpec(memory_space=pl.ANY),
        out_shape=jax.ShapeDtypeStruct((B, C + HIDDEN, HW), jnp.float32),
        scratch_shapes=[
            pltpu.VMEM((RING, NB, HIDDEN, HW), jnp.float32),
            pltpu.SemaphoreType.DMA((RING,)),
            pltpu.SemaphoreType.DMA((RING,)),
        ],
        compiler_params=pltpu.CompilerParams(
            dimension_semantics=("arbitrary",)),
    )(aux3, lat3)
    return out.reshape(B, C + HIDDEN, SPATIAL, SPATIAL)


# speedup vs baseline: 6.7620x; 6.7620x over previous
import jax
import jax.numpy as jnp
from jax.experimental import pallas as pl
from jax.experimental.pallas import tpu as pltpu

NBITS = 256
HIDDEN = 256
SPATIAL = 32
B = 128
C = 128
HW = SPATIAL * SPATIAL

NB = 8
GRID = B // NB


def _aux_body(msg_ref, we_ref, wo_ref, aux_ref):
    we = we_ref[...]
    d = wo_ref[...] - we
    base = jnp.sum(we, axis=0, keepdims=True)
    aux_ref[...] = jax.lax.dot_general(
        msg_ref[...], d, (((1,), (0,)), ((), ())),
        preferred_element_type=jnp.float32) + base


def _bcast_body(aux_ref, lat_ref, out_ref):
    out_ref[:, :C, :] = lat_ref[...]
    out_ref[:, C:, :] = jnp.broadcast_to(aux_ref[...], (NB, HIDDEN, HW))


def kernel(latents, msg, W):
    lat3 = latents.reshape(B, C, HW)
    msg_f = msg.astype(jnp.float32)
    we = W[0::2]
    wo = W[1::2]
    aux = pl.pallas_call(
        _aux_body,
        in_specs=[
            pl.BlockSpec((B, NBITS), lambda: (0, 0)),
            pl.BlockSpec((NBITS, HIDDEN), lambda: (0, 0)),
            pl.BlockSpec((NBITS, HIDDEN), lambda: (0, 0)),
        ],
        out_specs=pl.BlockSpec((B, HIDDEN), lambda: (0, 0)),
        out_shape=jax.ShapeDtypeStruct((B, HIDDEN), jnp.float32),
    )(msg_f, we, wo)
    aux3 = aux.reshape(B, HIDDEN, 1)
    out = pl.pallas_call(
        _bcast_body,
        grid=(GRID,),
        in_specs=[
            pl.BlockSpec((NB, HIDDEN, 1), lambda g: (g, 0, 0)),
            pl.BlockSpec((NB, C, HW), lambda g: (g, 0, 0)),
        ],
        out_specs=pl.BlockSpec((NB, C + HIDDEN, HW), lambda g: (g, 0, 0)),
        out_shape=jax.ShapeDtypeStruct((B, C + HIDDEN, HW), jnp.float32),
    )(aux3, lat3)
    return out.reshape(B, C + HIDDEN, SPATIAL, SPATIAL)
